# baseline (speedup 1.0000x reference)
import functools

import jax
import jax.numpy as jnp
from jax import lax
from jax.experimental import pallas as pl
from jax.experimental.pallas import tpu as pltpu
from jax.experimental.pallas import tpu_sc as plsc

VOCAB = 30
D = 1024
B = 32 * 1024

NC = 2
NS = 16
NW = NC * NS
B_PER_W = B // NW
GSIZE = 16
NGROUP = B_PER_W // GSIZE

_mesh = plsc.VectorSubcoreMesh(
    core_axis_name="c", subcore_axis_name="s", num_cores=NC, num_subcores=NS
)


@functools.partial(
    pl.kernel,
    out_type=jax.ShapeDtypeStruct((B, D), jnp.float32),
    mesh=_mesh,
    scratch_types=[
        pltpu.VMEM((B_PER_W,), jnp.int32),
        pltpu.VMEM((VOCAB, D), jnp.float32),
        pltpu.SemaphoreType.DMA,
    ],
)
def _embed(x_hbm, table_hbm, out_hbm, idx_v, table_v, wsem):
    wid = lax.axis_index("s") * NC + lax.axis_index("c")
    base = wid * B_PER_W
    pltpu.sync_copy(x_hbm.at[wid], idx_v)
    pltpu.sync_copy(table_hbm, table_v)


    def drain(n):
        pltpu.make_async_copy(
            table_v.at[pl.ds(0, n)], out_hbm.at[pl.ds(base, n)], wsem).wait()

    def issue_group(g):
        xv = idx_v[pl.ds(g * GSIZE, GSIZE)]
        for j in range(GSIZE):
            pltpu.async_copy(
                table_v.at[pl.ds(xv[j], 1)],
                out_hbm.at[pl.ds(base + g * GSIZE + j, 1)], wsem)

    issue_group(0)

    def body(g, carry):
        issue_group(g + 1)
        drain(GSIZE)
        return carry

    lax.fori_loop(0, NGROUP - 1, body, 0)
    drain(GSIZE)


@jax.jit
def kernel(x, table):
    x_r = x.reshape(NW, B_PER_W)
    out = _embed(x_r, table)
    return out.reshape(32, 1024, D)
